# baseline (speedup 1.0000x reference)
import functools

import jax
import jax.numpy as jnp
from jax import lax
from jax.experimental import pallas as pl
from jax.experimental.pallas import tpu as pltpu

_B, _N, _D = 16384, 7, 256
_C = 64
_BLK = 2048


def _fused_kernel(cards_ref, rank_ref, suit_ref, card_ref, w_ref, b_ref,
                  out_ref, m_scr):
    @pl.when(pl.program_id(0) == 0)
    def _build():
        row = lax.broadcasted_iota(jnp.int32, (_C, 1), 0)
        valid = row < 52
        oh_r = jnp.where(
            (row % 13 == lax.broadcasted_iota(jnp.int32, (_C, 16), 1)) & valid,
            1.0, 0.0)
        oh_s = jnp.where(
            (row // 13 == lax.broadcasted_iota(jnp.int32, (_C, 8), 1)) & valid,
            1.0, 0.0)
        rank_pad = jnp.concatenate(
            [rank_ref[...], jnp.zeros((3, _D), jnp.float32)], axis=0)
        suit_pad = jnp.concatenate(
            [suit_ref[...], jnp.zeros((4, _D), jnp.float32)], axis=0)
        card_pad = jnp.concatenate(
            [card_ref[...], jnp.zeros((12, _D), jnp.float32)], axis=0)
        t = (
            lax.dot_general(oh_r, rank_pad, (((1,), (0,)), ((), ())),
                            preferred_element_type=jnp.float32)
            + lax.dot_general(oh_s, suit_pad, (((1,), (0,)), ((), ())),
                              preferred_element_type=jnp.float32)
            + card_pad
        )
        m_scr[...] = lax.dot_general(
            t, w_ref[...], (((1,), (1,)), ((), ())),
            preferred_element_type=jnp.float32).astype(jnp.bfloat16)

    cards = cards_ref[...]
    bins = lax.broadcasted_iota(jnp.int32, (_BLK, _C), 1)
    counts = jnp.zeros((_BLK, _C), jnp.float32)
    for n in range(_N):
        counts += jnp.where(cards[:, n:n + 1] == bins, 1.0, 0.0)
    acc = lax.dot_general(
        counts.astype(jnp.bfloat16), m_scr[...], (((1,), (0,)), ((), ())),
        preferred_element_type=jnp.float32)
    out_ref[...] = jnp.maximum(acc + b_ref[...], 0.0)


def kernel(cards, rank_emb, suit_emb, card_emb, W, b):
    grid = (_B // _BLK,)
    return pl.pallas_call(
        _fused_kernel,
        grid=grid,
        in_specs=[
            pl.BlockSpec((_BLK, _N), lambda i: (i, 0)),
            pl.BlockSpec((13, _D), lambda i: (0, 0)),
            pl.BlockSpec((4, _D), lambda i: (0, 0)),
            pl.BlockSpec((52, _D), lambda i: (0, 0)),
            pl.BlockSpec((_D, _D), lambda i: (0, 0)),
            pl.BlockSpec((1, _D), lambda i: (0, 0)),
        ],
        out_specs=pl.BlockSpec((_BLK, _D), lambda i: (i, 0)),
        out_shape=jax.ShapeDtypeStruct((_B, _D), jnp.float32),
        scratch_shapes=[pltpu.VMEM((_C, _D), jnp.bfloat16)],
    )(cards, rank_emb, suit_emb, card_emb, W, b.reshape(1, _D))


# speedup vs baseline: 1.2625x; 1.2625x over previous
import functools

import jax
import jax.numpy as jnp
from jax import lax
from jax.experimental import pallas as pl
from jax.experimental.pallas import tpu as pltpu

_B, _N, _D = 16384, 7, 256
_C = 64
_BLK = 2048


def _fused_kernel(cards_ref, rank_ref, suit_ref, card_ref, w_ref, b_ref,
                  out_ref, m_scr):
    @pl.when(pl.program_id(0) == 0)
    def _build():
        row = lax.broadcasted_iota(jnp.int32, (_C, 1), 0)
        valid = row < 52
        oh_r = jnp.where(
            (row % 13 == lax.broadcasted_iota(jnp.int32, (_C, 16), 1)) & valid,
            1.0, 0.0)
        oh_s = jnp.where(
            (row // 13 == lax.broadcasted_iota(jnp.int32, (_C, 8), 1)) & valid,
            1.0, 0.0)
        rank_pad = jnp.concatenate(
            [rank_ref[...], jnp.zeros((3, _D), jnp.float32)], axis=0)
        suit_pad = jnp.concatenate(
            [suit_ref[...], jnp.zeros((4, _D), jnp.float32)], axis=0)
        card_pad = jnp.concatenate(
            [card_ref[...], jnp.zeros((12, _D), jnp.float32)], axis=0)
        t = (
            lax.dot_general(oh_r, rank_pad, (((1,), (0,)), ((), ())),
                            preferred_element_type=jnp.float32)
            + lax.dot_general(oh_s, suit_pad, (((1,), (0,)), ((), ())),
                              preferred_element_type=jnp.float32)
            + card_pad
        )
        m_scr[...] = lax.dot_general(
            t, w_ref[...], (((1,), (1,)), ((), ())),
            preferred_element_type=jnp.float32).astype(jnp.bfloat16)

    cards = cards_ref[...].astype(jnp.bfloat16)
    bins = lax.broadcasted_iota(jnp.int32, (_BLK, _C), 1).astype(jnp.bfloat16)
    counts = jnp.zeros((_BLK, _C), jnp.bfloat16)
    for n in range(_N):
        counts += jnp.where(cards[:, n:n + 1] == bins,
                            jnp.bfloat16(1.0), jnp.bfloat16(0.0))
    acc = lax.dot_general(
        counts, m_scr[...], (((1,), (0,)), ((), ())),
        preferred_element_type=jnp.float32)
    out_ref[...] = jnp.maximum(acc + b_ref[...], 0.0)


def kernel(cards, rank_emb, suit_emb, card_emb, W, b):
    grid = (_B // _BLK,)
    return pl.pallas_call(
        _fused_kernel,
        grid=grid,
        in_specs=[
            pl.BlockSpec((_BLK, _N), lambda i: (i, 0)),
            pl.BlockSpec((13, _D), lambda i: (0, 0)),
            pl.BlockSpec((4, _D), lambda i: (0, 0)),
            pl.BlockSpec((52, _D), lambda i: (0, 0)),
            pl.BlockSpec((_D, _D), lambda i: (0, 0)),
            pl.BlockSpec((1, _D), lambda i: (0, 0)),
        ],
        out_specs=pl.BlockSpec((_BLK, _D), lambda i: (i, 0)),
        out_shape=jax.ShapeDtypeStruct((_B, _D), jnp.float32),
        scratch_shapes=[pltpu.VMEM((_C, _D), jnp.bfloat16)],
    )(cards, rank_emb, suit_emb, card_emb, W, b.reshape(1, _D))


# speedup vs baseline: 1.3378x; 1.0596x over previous
import functools

import jax
import jax.numpy as jnp
from jax import lax
from jax.experimental import pallas as pl
from jax.experimental.pallas import tpu as pltpu

_B, _N, _D = 16384, 7, 256
_C = 64
_BLK = 4096


def _fused_kernel(cards_ref, rank_ref, suit_ref, card_ref, w_ref, b_ref,
                  out_ref, m_scr):
    @pl.when(pl.program_id(0) == 0)
    def _build():
        row = lax.broadcasted_iota(jnp.int32, (_C, 1), 0)
        valid = row < 52
        oh_r = jnp.where(
            (row % 13 == lax.broadcasted_iota(jnp.int32, (_C, 16), 1)) & valid,
            1.0, 0.0)
        oh_s = jnp.where(
            (row // 13 == lax.broadcasted_iota(jnp.int32, (_C, 8), 1)) & valid,
            1.0, 0.0)
        rank_pad = jnp.concatenate(
            [rank_ref[...], jnp.zeros((3, _D), jnp.float32)], axis=0)
        suit_pad = jnp.concatenate(
            [suit_ref[...], jnp.zeros((4, _D), jnp.float32)], axis=0)
        card_pad = jnp.concatenate(
            [card_ref[...], jnp.zeros((12, _D), jnp.float32)], axis=0)
        t = (
            lax.dot_general(oh_r, rank_pad, (((1,), (0,)), ((), ())),
                            preferred_element_type=jnp.float32)
            + lax.dot_general(oh_s, suit_pad, (((1,), (0,)), ((), ())),
                              preferred_element_type=jnp.float32)
            + card_pad
        )
        m_scr[...] = lax.dot_general(
            t, w_ref[...], (((1,), (1,)), ((), ())),
            preferred_element_type=jnp.float32).astype(jnp.bfloat16)

    cards = cards_ref[...].astype(jnp.bfloat16)
    bins = lax.broadcasted_iota(jnp.int32, (_BLK, _C), 1).astype(jnp.bfloat16)
    counts = jnp.zeros((_BLK, _C), jnp.bfloat16)
    for n in range(_N):
        counts += jnp.where(cards[:, n:n + 1] == bins,
                            jnp.bfloat16(1.0), jnp.bfloat16(0.0))
    acc = lax.dot_general(
        counts, m_scr[...], (((1,), (0,)), ((), ())),
        preferred_element_type=jnp.float32)
    out_ref[...] = jnp.maximum(acc + b_ref[...], 0.0)


def kernel(cards, rank_emb, suit_emb, card_emb, W, b):
    grid = (_B // _BLK,)
    return pl.pallas_call(
        _fused_kernel,
        grid=grid,
        in_specs=[
            pl.BlockSpec((_BLK, _N), lambda i: (i, 0)),
            pl.BlockSpec((13, _D), lambda i: (0, 0)),
            pl.BlockSpec((4, _D), lambda i: (0, 0)),
            pl.BlockSpec((52, _D), lambda i: (0, 0)),
            pl.BlockSpec((_D, _D), lambda i: (0, 0)),
            pl.BlockSpec((1, _D), lambda i: (0, 0)),
        ],
        out_specs=pl.BlockSpec((_BLK, _D), lambda i: (i, 0)),
        out_shape=jax.ShapeDtypeStruct((_B, _D), jnp.float32),
        scratch_shapes=[pltpu.VMEM((_C, _D), jnp.bfloat16)],
    )(cards, rank_emb, suit_emb, card_emb, W, b.reshape(1, _D))
